# baseline (speedup 1.0000x reference)
import functools

import jax
import jax.numpy as jnp
from jax.experimental import pallas as pl
from jax.experimental.pallas import tpu as pltpu

_EPS = 1e-12


def _mm(a, b):
    return jax.lax.dot_general(
        a.astype(jnp.bfloat16), b.astype(jnp.bfloat16),
        (((1,), (1,)), ((), ())),
        preferred_element_type=jnp.float32,
        precision=jax.lax.Precision.DEFAULT)


def _unsortable(m):
    bits = jnp.where(m < 0, m ^ jnp.int32(0x7FFFFFFF), m)
    return jax.lax.bitcast_convert_type(bits, jnp.float32)


def _sortable(x):
    bits = jax.lax.bitcast_convert_type(x, jnp.int32)
    return jnp.where(bits < 0, bits ^ jnp.int32(0x7FFFFFFF), bits)


def _fused_kernel(x_ref, w_ref, b_ref, q_ref,
                  h_ref, div_ref, cov_ref,
                  s_ref, st_ref, sb_ref,
                  *, k, n, r, nb):
    i = pl.program_id(0)

    @pl.when(i < nb)
    def _scores():
        x = x_ref[...]
        y = _mm(x, w_ref[...])
        y = y + b_ref[...]
        nrm = jnp.sqrt(jnp.sum(y * y, axis=1, keepdims=True))
        node = y / jnp.maximum(nrm, _EPS)
        q = q_ref[...]
        qn = q / jnp.maximum(
            jnp.sqrt(jnp.sum(q * q, axis=1, keepdims=True)), _EPS)
        sT = _mm(qn, node)
        col = jax.lax.broadcasted_iota(jnp.int32, sT.shape, 1) + i * r
        sT = jnp.where(col < n, sT, -jnp.inf)
        s_ref[:, pl.ds(i * r, r)] = sT
        sb_ref[:, pl.ds(i * r, r)] = jax.lax.shift_right_arithmetic(
            _sortable(sT), 16).astype(jnp.int16)

    @pl.when(i == nb)
    def _select():
        sT = s_ref[...]
        mx = jnp.max(sT, axis=1, keepdims=True)

        shape = mx.shape
        lk = jnp.float32(jnp.log(float(k)))
        sb = sb_ref[...]

        lo0 = jnp.full(shape, _sortable(jnp.float32(-1.5)) >> 16)
        hi0 = jnp.full(shape, _sortable(jnp.float32(1.5)) >> 16)
        cl0 = jnp.full(shape, jnp.int32(n))
        ch0 = jnp.zeros(shape, jnp.int32)
        tf0 = jnp.zeros(shape, jnp.int32)
        fnd0 = jnp.zeros(shape, jnp.int32)

        def cond1(st):
            it, lo, hi, cl, ch, tf, fnd = st
            act = ((hi - lo) > 1) & (fnd == 0)
            return (it < 40) & (jnp.max(act.astype(jnp.int32)) == 1)

        def body1(st):
            it, lo, hi, cl, ch, tf, fnd = st
            vl = _unsortable(jnp.left_shift(lo, 16))
            vh = _unsortable(jnp.left_shift(hi, 16))
            ll = jnp.log(jnp.maximum(cl.astype(jnp.float32), 0.5))
            lh = jnp.log(jnp.maximum(ch.astype(jnp.float32), 0.5))
            frac = (ll - lk) / jnp.maximum(ll - lh, 1e-6)
            tm = jax.lax.shift_right_arithmetic(
                _sortable(vl + (vh - vl) * frac), 16)
            mid = lo + jax.lax.shift_right_arithmetic(hi - lo, 1)
            tm = jnp.where((it >= 12) & ((it % 2) == 1), mid, tm)
            tm = jnp.clip(tm, lo + 1, hi - 1)
            cnt = jnp.sum((sb >= tm.astype(jnp.int16)).astype(jnp.int32),
                          axis=1, keepdims=True)
            ge = cnt >= k
            nlo = jnp.where(ge, tm, lo)
            ncl = jnp.where(ge, cnt, cl)
            nhi = jnp.where(ge, hi, tm)
            nch = jnp.where(ge, ch, cnt)
            hit = (cnt == k) & (fnd == 0)
            tf = jnp.where(hit, jnp.left_shift(tm, 16), tf)
            fnd = jnp.where(hit, jnp.int32(1), fnd)
            return (it + 1, nlo, nhi, ncl, nch, tf, fnd)

        st = jax.lax.while_loop(
            cond1, body1, (jnp.int32(0), lo0, hi0, cl0, ch0, tf0, fnd0))
        _, lo1, hi1, cl1, ch1, tf1, fnd1 = st

        lo2 = jnp.left_shift(lo1, 16)
        hi2 = jnp.left_shift(hi1, 16)

        def cond2(st):
            it, lo, hi, cl, ch, tf, fnd = st
            return (it < 48) & (jnp.min(fnd) == 0)

        def body2(st):
            it, lo, hi, cl, ch, tf, fnd = st
            vl = _unsortable(lo)
            vh = _unsortable(hi)
            ll = jnp.log(jnp.maximum(cl.astype(jnp.float32), 0.5))
            lh = jnp.log(jnp.maximum(ch.astype(jnp.float32), 0.5))
            frac = (ll - lk) / jnp.maximum(ll - lh, 1e-6)
            tm = _sortable(vl + (vh - vl) * frac)
            mid = lo + jax.lax.shift_right_arithmetic(hi - lo, 1)
            tm = jnp.where((it >= 8) & ((it % 2) == 1), mid, tm)
            tm = jnp.clip(tm, lo + 1, hi - 1)
            cnt = jnp.sum((sT >= _unsortable(tm)).astype(jnp.int32),
                          axis=1, keepdims=True)
            ge = cnt >= k
            nlo = jnp.where(ge, tm, lo)
            ncl = jnp.where(ge, cnt, cl)
            nhi = jnp.where(ge, hi, tm)
            nch = jnp.where(ge, ch, cnt)
            new = (fnd == 0)
            hit = (cnt == k) & new
            conv = ((nhi - nlo) <= 1) & new & jnp.logical_not(hit)
            tf = jnp.where(hit, tm, jnp.where(conv, nlo, tf))
            fnd = jnp.where(hit | conv, jnp.int32(1), fnd)
            return (it + 1, nlo, nhi, ncl, nch, tf, fnd)

        st = jax.lax.while_loop(
            cond2, body2, (jnp.int32(0), lo2, hi2, cl1, ch1, tf1, fnd1))
        t = _unsortable(st[5])

        sel = sT >= t
        ex = jnp.where(sel, jnp.exp(sT - mx), 0.0)
        z = jnp.sum(ex, axis=1, keepdims=True)
        s_ref[...] = ex

        st_ref[:, 0:1] = t
        st_ref[:, 1:2] = mx
        st_ref[:, 2:3] = z

        valid = (jax.lax.broadcasted_iota(jnp.int32, (1, sT.shape[1]), 1) < n)
        row_cnt = jnp.sum(sel.astype(jnp.float32), axis=0, keepdims=True)
        cov = jnp.sum(jnp.where(valid, jnp.maximum(1.0 - row_cnt, 0.0), 0.0)) / n
        cov_ref[...] = jnp.reshape(cov, (1, 1))

        q = q_ref[...]
        qn = q / jnp.maximum(
            jnp.sqrt(jnp.sum(q * q, axis=1, keepdims=True)), _EPS)
        qs = _mm(qn, qn)
        e = qs.shape[0]
        eye = (jax.lax.broadcasted_iota(jnp.int32, (e, e), 0) ==
               jax.lax.broadcasted_iota(jnp.int32, (e, e), 1)).astype(jnp.float32)
        denom = jnp.maximum(jnp.sum(1.0 - eye), 1.0)
        div = jnp.sum(((qs - eye) ** 2) * (1.0 - eye)) / denom
        div_ref[...] = jnp.reshape(div, (1, 1))

    @pl.when(i > nb)
    def _emit():
        j = i - (nb + 1)
        ex = s_ref[:, pl.ds(j * r, r)]
        z = st_ref[:, 2:3]
        h_ref[...] = (ex / z).T


def kernel(node_state, W, b, edge_queries):
    n, hd = node_state.shape
    e = edge_queries.shape[0]
    k = min(n, max(8, min(64, int(n * 0.1))))

    r = 4096
    nb = pl.cdiv(n, r)
    grid = 2 * nb + 1

    h, div, cov = pl.pallas_call(
        functools.partial(_fused_kernel, k=k, n=n, r=r, nb=nb),
        grid=(grid,),
        in_specs=[
            pl.BlockSpec((r, hd), lambda i: (jnp.minimum(i, nb - 1), 0)),
            pl.BlockSpec((hd, hd), lambda i: (0, 0)),
            pl.BlockSpec((1, hd), lambda i: (0, 0)),
            pl.BlockSpec((e, hd), lambda i: (0, 0)),
        ],
        out_specs=[
            pl.BlockSpec((r, e), lambda i: (jnp.maximum(i - (nb + 1), 0), 0)),
            pl.BlockSpec((1, 1), lambda i: (0, 0)),
            pl.BlockSpec((1, 1), lambda i: (0, 0)),
        ],
        out_shape=[
            jax.ShapeDtypeStruct((n, e), jnp.float32),
            jax.ShapeDtypeStruct((1, 1), jnp.float32),
            jax.ShapeDtypeStruct((1, 1), jnp.float32),
        ],
        scratch_shapes=[
            pltpu.VMEM((e, nb * r), jnp.float32),
            pltpu.VMEM((e, 128), jnp.float32),
            pltpu.VMEM((e, nb * r), jnp.int16),
        ],
    )(node_state, W, b.reshape(1, hd), edge_queries)

    return (h, div[0, 0], cov[0, 0])


# speedup vs baseline: 1.0875x; 1.0875x over previous
import functools

import jax
import jax.numpy as jnp
from jax.experimental import pallas as pl
from jax.experimental.pallas import tpu as pltpu

_EPS = 1e-12


def _mm(a, b):
    return jax.lax.dot_general(
        a.astype(jnp.bfloat16), b.astype(jnp.bfloat16),
        (((1,), (1,)), ((), ())),
        preferred_element_type=jnp.float32,
        precision=jax.lax.Precision.DEFAULT)


def _unsortable(m):
    bits = jnp.where(m < 0, m ^ jnp.int32(0x7FFFFFFF), m)
    return jax.lax.bitcast_convert_type(bits, jnp.float32)


def _sortable(x):
    bits = jax.lax.bitcast_convert_type(x, jnp.int32)
    return jnp.where(bits < 0, bits ^ jnp.int32(0x7FFFFFFF), bits)


def _fused_kernel(x_ref, w_ref, b_ref, q_ref,
                  h_ref, div_ref, cov_ref,
                  s_ref, st_ref,
                  *, k, n, r, nb):
    i = pl.program_id(0)

    @pl.when(i < nb)
    def _scores():
        x = x_ref[...]
        y = _mm(x, w_ref[...])
        y = y + b_ref[...]
        nrm = jnp.sqrt(jnp.sum(y * y, axis=1, keepdims=True))
        node = y / jnp.maximum(nrm, _EPS)
        q = q_ref[...]
        qn = q / jnp.maximum(
            jnp.sqrt(jnp.sum(q * q, axis=1, keepdims=True)), _EPS)
        sT = _mm(qn, node)
        col = jax.lax.broadcasted_iota(jnp.int32, sT.shape, 1) + i * r
        sT = jnp.where(col < n, sT, -jnp.inf)
        s_ref[:, pl.ds(i * r, r)] = sT

    @pl.when(i == nb)
    def _select():
        sT = s_ref[...]
        mx = jnp.max(sT, axis=1, keepdims=True)

        shape = mx.shape
        lo0 = jnp.full(shape, _sortable(jnp.float32(-1.5)))
        hi0 = jnp.full(shape, _sortable(jnp.float32(1.5)))
        cl0 = jnp.full(shape, jnp.int32(n))
        ch0 = jnp.zeros(shape, jnp.int32)
        tf0 = jnp.zeros(shape, jnp.int32)
        fnd0 = jnp.zeros(shape, jnp.int32)
        lk = jnp.float32(jnp.log(float(k)))

        def cond(st):
            it, lo, hi, cl, ch, tf, fnd = st
            return (it < 80) & (jnp.min(fnd) == 0)

        def body(st):
            it, lo, hi, cl, ch, tf, fnd = st
            vl = _unsortable(lo)
            vh = _unsortable(hi)
            ll = jnp.log(jnp.maximum(cl.astype(jnp.float32), 0.5))
            lh = jnp.log(jnp.maximum(ch.astype(jnp.float32), 0.5))
            frac = (ll - lk) / jnp.maximum(ll - lh, 1e-6)
            tm = _sortable(vl + (vh - vl) * frac)
            mid = lo + jax.lax.shift_right_arithmetic(hi - lo, 1)
            tm = jnp.where((it >= 16) & ((it % 2) == 1), mid, tm)
            tm = jnp.clip(tm, lo + 1, hi - 1)
            cnt = jnp.sum((sT >= _unsortable(tm)).astype(jnp.int32),
                          axis=1, keepdims=True)
            ge = cnt >= k
            nlo = jnp.where(ge, tm, lo)
            ncl = jnp.where(ge, cnt, cl)
            nhi = jnp.where(ge, hi, tm)
            nch = jnp.where(ge, ch, cnt)
            new = (fnd == 0)
            hit = (cnt == k) & new
            conv = ((nhi - nlo) <= 1) & new & jnp.logical_not(hit)
            tf = jnp.where(hit, tm, jnp.where(conv, nlo, tf))
            fnd = jnp.where(hit | conv, jnp.int32(1), fnd)
            return (it + 1, nlo, nhi, ncl, nch, tf, fnd)

        st = jax.lax.while_loop(cond, body,
                                (jnp.int32(0), lo0, hi0, cl0, ch0, tf0, fnd0))
        t = _unsortable(st[5])

        sel = sT >= t
        ex = jnp.where(sel, jnp.exp(sT - mx), 0.0)
        z = jnp.sum(ex, axis=1, keepdims=True)
        s_ref[...] = ex

        st_ref[:, 0:1] = t
        st_ref[:, 1:2] = mx
        st_ref[:, 2:3] = z

        valid = (jax.lax.broadcasted_iota(jnp.int32, (1, sT.shape[1]), 1) < n)
        row_cnt = jnp.sum(sel.astype(jnp.float32), axis=0, keepdims=True)
        cov = jnp.sum(jnp.where(valid, jnp.maximum(1.0 - row_cnt, 0.0), 0.0)) / n
        cov_ref[...] = jnp.reshape(cov, (1, 1))

        q = q_ref[...]
        qn = q / jnp.maximum(
            jnp.sqrt(jnp.sum(q * q, axis=1, keepdims=True)), _EPS)
        qs = _mm(qn, qn)
        e = qs.shape[0]
        eye = (jax.lax.broadcasted_iota(jnp.int32, (e, e), 0) ==
               jax.lax.broadcasted_iota(jnp.int32, (e, e), 1)).astype(jnp.float32)
        denom = jnp.maximum(jnp.sum(1.0 - eye), 1.0)
        div = jnp.sum(((qs - eye) ** 2) * (1.0 - eye)) / denom
        div_ref[...] = jnp.reshape(div, (1, 1))

    @pl.when(i > nb)
    def _emit():
        j = i - (nb + 1)
        ex = s_ref[:, pl.ds(j * r, r)]
        z = st_ref[:, 2:3]
        h_ref[...] = (ex / z).T


def kernel(node_state, W, b, edge_queries):
    n, hd = node_state.shape
    e = edge_queries.shape[0]
    k = min(n, max(8, min(64, int(n * 0.1))))

    r = 5120
    nb = pl.cdiv(n, r)
    grid = 2 * nb + 1

    h, div, cov = pl.pallas_call(
        functools.partial(_fused_kernel, k=k, n=n, r=r, nb=nb),
        grid=(grid,),
        in_specs=[
            pl.BlockSpec((r, hd), lambda i: (jnp.minimum(i, nb - 1), 0)),
            pl.BlockSpec((hd, hd), lambda i: (0, 0)),
            pl.BlockSpec((1, hd), lambda i: (0, 0)),
            pl.BlockSpec((e, hd), lambda i: (0, 0)),
        ],
        out_specs=[
            pl.BlockSpec((r, e), lambda i: (jnp.maximum(i - (nb + 1), 0), 0)),
            pl.BlockSpec((1, 1), lambda i: (0, 0)),
            pl.BlockSpec((1, 1), lambda i: (0, 0)),
        ],
        out_shape=[
            jax.ShapeDtypeStruct((n, e), jnp.float32),
            jax.ShapeDtypeStruct((1, 1), jnp.float32),
            jax.ShapeDtypeStruct((1, 1), jnp.float32),
        ],
        scratch_shapes=[
            pltpu.VMEM((e, nb * r), jnp.float32),
            pltpu.VMEM((e, 128), jnp.float32),
        ],
    )(node_state, W, b.reshape(1, hd), edge_queries)

    return (h, div[0, 0], cov[0, 0])


# speedup vs baseline: 1.0916x; 1.0038x over previous
import functools

import jax
import jax.numpy as jnp
from jax.experimental import pallas as pl
from jax.experimental.pallas import tpu as pltpu

_EPS = 1e-12


def _mm(a, b):
    return jax.lax.dot_general(
        a.astype(jnp.bfloat16), b.astype(jnp.bfloat16),
        (((1,), (1,)), ((), ())),
        preferred_element_type=jnp.float32,
        precision=jax.lax.Precision.DEFAULT)


def _unsortable(m):
    bits = jnp.where(m < 0, m ^ jnp.int32(0x7FFFFFFF), m)
    return jax.lax.bitcast_convert_type(bits, jnp.float32)


def _sortable(x):
    bits = jax.lax.bitcast_convert_type(x, jnp.int32)
    return jnp.where(bits < 0, bits ^ jnp.int32(0x7FFFFFFF), bits)


def _fused_kernel(x_ref, w_ref, b_ref, q_ref,
                  h_ref, div_ref, cov_ref,
                  s_ref, st_ref,
                  *, k, n, r, nb):
    i = pl.program_id(0)

    @pl.when(i < nb)
    def _scores():
        x = x_ref[...]
        y = _mm(x, w_ref[...])
        y = y + b_ref[...]
        nrm = jnp.sqrt(jnp.sum(y * y, axis=1, keepdims=True))
        node = y / jnp.maximum(nrm, _EPS)
        q = q_ref[...]
        qn = q / jnp.maximum(
            jnp.sqrt(jnp.sum(q * q, axis=1, keepdims=True)), _EPS)
        sT = _mm(qn, node)
        col = jax.lax.broadcasted_iota(jnp.int32, sT.shape, 1) + i * r
        sT = jnp.where(col < n, sT, -jnp.inf)
        s_ref[:, pl.ds(i * r, r)] = sT

    @pl.when(i == nb)
    def _select():
        sT = s_ref[...]
        mx = jnp.max(sT, axis=1, keepdims=True)

        shape = mx.shape
        lo0 = jnp.full(shape, _sortable(jnp.float32(-1.5)))
        hi0 = jnp.full(shape, _sortable(jnp.float32(1.5)))
        cl0 = jnp.full(shape, jnp.int32(n))
        ch0 = jnp.zeros(shape, jnp.int32)
        tf0 = jnp.zeros(shape, jnp.int32)
        fnd0 = jnp.zeros(shape, jnp.int32)
        lk = jnp.float32(jnp.log(float(k)))

        def cond(st):
            it, lo, hi, cl, ch, tf, fnd = st
            return (it < 80) & (jnp.min(fnd) == 0)

        def body(st):
            it, lo, hi, cl, ch, tf, fnd = st
            vl = _unsortable(lo)
            vh = _unsortable(hi)
            ll = jnp.log(jnp.maximum(cl.astype(jnp.float32), 0.5))
            lh = jnp.log(jnp.maximum(ch.astype(jnp.float32), 0.5))
            frac = (ll - lk) / jnp.maximum(ll - lh, 1e-6)
            tm = _sortable(vl + (vh - vl) * frac)
            mid = lo + jax.lax.shift_right_arithmetic(hi - lo, 1)
            tm = jnp.where((it >= 16) & ((it % 2) == 1), mid, tm)
            tm = jnp.clip(tm, lo + 1, hi - 1)
            cnt = jnp.sum((sT >= _unsortable(tm)).astype(jnp.int32),
                          axis=1, keepdims=True)
            ge = cnt >= k
            nlo = jnp.where(ge, tm, lo)
            ncl = jnp.where(ge, cnt, cl)
            nhi = jnp.where(ge, hi, tm)
            nch = jnp.where(ge, ch, cnt)
            new = (fnd == 0)
            hit = (cnt == k) & new
            conv = ((nhi - nlo) <= 1) & new & jnp.logical_not(hit)
            tf = jnp.where(hit, tm, jnp.where(conv, nlo, tf))
            fnd = jnp.where(hit | conv, jnp.int32(1), fnd)
            return (it + 1, nlo, nhi, ncl, nch, tf, fnd)

        st = jax.lax.while_loop(cond, body,
                                (jnp.int32(0), lo0, hi0, cl0, ch0, tf0, fnd0))
        t = _unsortable(st[5])

        sel = sT >= t
        ex = jnp.where(sel, jnp.exp(sT - mx), 0.0)
        z = jnp.sum(ex, axis=1, keepdims=True)
        s_ref[...] = ex

        st_ref[:, 0:1] = t
        st_ref[:, 1:2] = mx
        st_ref[:, 2:3] = z

        valid = (jax.lax.broadcasted_iota(jnp.int32, (1, sT.shape[1]), 1) < n)
        row_cnt = jnp.sum(sel.astype(jnp.float32), axis=0, keepdims=True)
        cov = jnp.sum(jnp.where(valid, jnp.maximum(1.0 - row_cnt, 0.0), 0.0)) / n
        cov_ref[...] = jnp.reshape(cov, (1, 1))

        q = q_ref[...]
        qn = q / jnp.maximum(
            jnp.sqrt(jnp.sum(q * q, axis=1, keepdims=True)), _EPS)
        qs = _mm(qn, qn)
        e = qs.shape[0]
        eye = (jax.lax.broadcasted_iota(jnp.int32, (e, e), 0) ==
               jax.lax.broadcasted_iota(jnp.int32, (e, e), 1)).astype(jnp.float32)
        denom = jnp.maximum(jnp.sum(1.0 - eye), 1.0)
        div = jnp.sum(((qs - eye) ** 2) * (1.0 - eye)) / denom
        div_ref[...] = jnp.reshape(div, (1, 1))

    @pl.when(i > nb)
    def _emit():
        j = i - (nb + 1)
        ex = s_ref[:, pl.ds(j * r, r)]
        z = st_ref[:, 2:3]
        h_ref[...] = (ex / z).T


def kernel(node_state, W, b, edge_queries):
    n, hd = node_state.shape
    e = edge_queries.shape[0]
    k = min(n, max(8, min(64, int(n * 0.1))))

    r = 6400
    nb = pl.cdiv(n, r)
    grid = 2 * nb + 1

    h, div, cov = pl.pallas_call(
        functools.partial(_fused_kernel, k=k, n=n, r=r, nb=nb),
        grid=(grid,),
        in_specs=[
            pl.BlockSpec((r, hd), lambda i: (jnp.minimum(i, nb - 1), 0)),
            pl.BlockSpec((hd, hd), lambda i: (0, 0)),
            pl.BlockSpec((1, hd), lambda i: (0, 0)),
            pl.BlockSpec((e, hd), lambda i: (0, 0)),
        ],
        out_specs=[
            pl.BlockSpec((r, e), lambda i: (jnp.maximum(i - (nb + 1), 0), 0)),
            pl.BlockSpec((1, 1), lambda i: (0, 0)),
            pl.BlockSpec((1, 1), lambda i: (0, 0)),
        ],
        out_shape=[
            jax.ShapeDtypeStruct((n, e), jnp.float32),
            jax.ShapeDtypeStruct((1, 1), jnp.float32),
            jax.ShapeDtypeStruct((1, 1), jnp.float32),
        ],
        scratch_shapes=[
            pltpu.VMEM((e, nb * r), jnp.float32),
            pltpu.VMEM((e, 128), jnp.float32),
        ],
    )(node_state, W, b.reshape(1, hd), edge_queries)

    return (h, div[0, 0], cov[0, 0])


# speedup vs baseline: 1.0924x; 1.0007x over previous
import functools

import jax
import jax.numpy as jnp
from jax.experimental import pallas as pl
from jax.experimental.pallas import tpu as pltpu

_EPS = 1e-12


def _mm(a, b):
    return jax.lax.dot_general(
        a.astype(jnp.bfloat16), b.astype(jnp.bfloat16),
        (((1,), (1,)), ((), ())),
        preferred_element_type=jnp.float32,
        precision=jax.lax.Precision.DEFAULT)


def _unsortable(m):
    bits = jnp.where(m < 0, m ^ jnp.int32(0x7FFFFFFF), m)
    return jax.lax.bitcast_convert_type(bits, jnp.float32)


def _sortable(x):
    bits = jax.lax.bitcast_convert_type(x, jnp.int32)
    return jnp.where(bits < 0, bits ^ jnp.int32(0x7FFFFFFF), bits)


def _fused_kernel(x_ref, w_ref, b_ref, q_ref,
                  h_ref, div_ref, cov_ref,
                  s_ref, st_ref,
                  *, k, n, r, nb):
    i = pl.program_id(0)

    @pl.when(i < nb)
    def _scores():
        x = x_ref[...]
        y = _mm(x, w_ref[...])
        y = y + b_ref[...]
        nrm = jnp.sqrt(jnp.sum(y * y, axis=1, keepdims=True))
        node = y / jnp.maximum(nrm, _EPS)
        q = q_ref[...]
        qn = q / jnp.maximum(
            jnp.sqrt(jnp.sum(q * q, axis=1, keepdims=True)), _EPS)
        sT = _mm(qn, node)
        col = jax.lax.broadcasted_iota(jnp.int32, sT.shape, 1) + i * r
        sT = jnp.where(col < n, sT, -jnp.inf)
        s_ref[:, pl.ds(i * r, r)] = sT

    @pl.when(i == nb)
    def _select():
        sT = s_ref[...]
        mx = jnp.max(sT, axis=1, keepdims=True)

        shape = mx.shape
        lo0 = jnp.full(shape, _sortable(jnp.float32(-1.5)))
        hi0 = jnp.full(shape, _sortable(jnp.float32(1.5)))
        cl0 = jnp.full(shape, jnp.int32(n))
        ch0 = jnp.zeros(shape, jnp.int32)
        tf0 = jnp.zeros(shape, jnp.int32)
        fnd0 = jnp.zeros(shape, jnp.int32)
        lk = jnp.float32(jnp.log(float(k)))

        def cond(st):
            it, lo, hi, cl, ch, tf, fnd = st
            return (it < 80) & (jnp.min(fnd) == 0)

        def body(st):
            it, lo, hi, cl, ch, tf, fnd = st
            vl = _unsortable(lo)
            vh = _unsortable(hi)
            ll = jnp.log(jnp.maximum(cl.astype(jnp.float32), 0.5))
            lh = jnp.log(jnp.maximum(ch.astype(jnp.float32), 0.5))
            frac = (ll - lk) / jnp.maximum(ll - lh, 1e-6)
            tm = _sortable(vl + (vh - vl) * frac)
            mid = lo + jax.lax.shift_right_arithmetic(hi - lo, 1)
            tm = jnp.where((it >= 16) & ((it % 2) == 1), mid, tm)
            tm = jnp.clip(tm, lo + 1, hi - 1)
            cnt = jnp.sum((sT >= _unsortable(tm)).astype(jnp.int32),
                          axis=1, keepdims=True)
            ge = cnt >= k
            nlo = jnp.where(ge, tm, lo)
            ncl = jnp.where(ge, cnt, cl)
            nhi = jnp.where(ge, hi, tm)
            nch = jnp.where(ge, ch, cnt)
            new = (fnd == 0)
            hit = (cnt == k) & new
            conv = ((nhi - nlo) <= 1) & new & jnp.logical_not(hit)
            tf = jnp.where(hit, tm, jnp.where(conv, nlo, tf))
            fnd = jnp.where(hit | conv, jnp.int32(1), fnd)
            return (it + 1, nlo, nhi, ncl, nch, tf, fnd)

        st = jax.lax.while_loop(cond, body,
                                (jnp.int32(0), lo0, hi0, cl0, ch0, tf0, fnd0))
        t = _unsortable(st[5])

        sel = sT >= t
        ex = jnp.where(sel, jnp.exp(sT - mx), 0.0)
        z = jnp.sum(ex, axis=1, keepdims=True)
        s_ref[...] = ex

        st_ref[:, 0:1] = t
        st_ref[:, 1:2] = mx
        st_ref[:, 2:3] = z
        st_ref[:, 3:4] = 1.0 / z

        valid = (jax.lax.broadcasted_iota(jnp.int32, (1, sT.shape[1]), 1) < n)
        row_cnt = jnp.sum(sel.astype(jnp.float32), axis=0, keepdims=True)
        cov = jnp.sum(jnp.where(valid, jnp.maximum(1.0 - row_cnt, 0.0), 0.0)) / n
        cov_ref[...] = jnp.reshape(cov, (1, 1))

        q = q_ref[...]
        qn = q / jnp.maximum(
            jnp.sqrt(jnp.sum(q * q, axis=1, keepdims=True)), _EPS)
        qs = _mm(qn, qn)
        e = qs.shape[0]
        eye = (jax.lax.broadcasted_iota(jnp.int32, (e, e), 0) ==
               jax.lax.broadcasted_iota(jnp.int32, (e, e), 1)).astype(jnp.float32)
        denom = jnp.maximum(jnp.sum(1.0 - eye), 1.0)
        div = jnp.sum(((qs - eye) ** 2) * (1.0 - eye)) / denom
        div_ref[...] = jnp.reshape(div, (1, 1))

    @pl.when(i > nb)
    def _emit():
        j = i - (nb + 1)
        ex = s_ref[:, pl.ds(j * r, r)]
        iz = st_ref[:, 3:4]
        h_ref[...] = (ex * iz).T


def kernel(node_state, W, b, edge_queries):
    n, hd = node_state.shape
    e = edge_queries.shape[0]
    k = min(n, max(8, min(64, int(n * 0.1))))

    r = 6400
    nb = pl.cdiv(n, r)
    grid = 2 * nb + 1

    h, div, cov = pl.pallas_call(
        functools.partial(_fused_kernel, k=k, n=n, r=r, nb=nb),
        grid=(grid,),
        in_specs=[
            pl.BlockSpec((r, hd), lambda i: (jnp.minimum(i, nb - 1), 0)),
            pl.BlockSpec((hd, hd), lambda i: (0, 0)),
            pl.BlockSpec((1, hd), lambda i: (0, 0)),
            pl.BlockSpec((e, hd), lambda i: (0, 0)),
        ],
        out_specs=[
            pl.BlockSpec((r, e), lambda i: (jnp.maximum(i - (nb + 1), 0), 0)),
            pl.BlockSpec((1, 1), lambda i: (0, 0)),
            pl.BlockSpec((1, 1), lambda i: (0, 0)),
        ],
        out_shape=[
            jax.ShapeDtypeStruct((n, e), jnp.float32),
            jax.ShapeDtypeStruct((1, 1), jnp.float32),
            jax.ShapeDtypeStruct((1, 1), jnp.float32),
        ],
        scratch_shapes=[
            pltpu.VMEM((e, nb * r), jnp.float32),
            pltpu.VMEM((e, 128), jnp.float32),
        ],
    )(node_state, W, b.reshape(1, hd), edge_queries)

    return (h, div[0, 0], cov[0, 0])
